# baseline (speedup 1.0000x reference)
import jax
import jax.numpy as jnp
from jax import lax
from jax.experimental import pallas as pl
from jax.experimental.pallas import tpu as pltpu
from jax.experimental.pallas import tpu_sc as plsc

B, S, D = 4, 2048, 1024
NC, NS = 2, 16
NW = NC * NS
S_PER_W = S // NW
R = 8
SUBS = S_PER_W // R
NCHUNKS = B * SUBS
NBUF = 6
LANES = 16
VREGS_PER_ROW = D // LANES
VREGS_PER_CHUNK = R * VREGS_PER_ROW


def _sc_body(x_hbm, pt_hbm, out_hbm, pe_buf, *rest):
    xbufs = rest[:NBUF]
    in_s = rest[NBUF:2 * NBUF]
    out_s = rest[2 * NBUF:3 * NBUF]
    pe_sems = rest[3 * NBUF:]
    cid = lax.axis_index("c")
    sid = lax.axis_index("s")
    wid = sid * NC + cid
    s0 = wid * S_PER_W
    bufs = tuple(xbufs)
    in_sems = tuple(in_s)
    out_sems = tuple(out_s)

    def loc(i):
        b, sub = divmod(i, SUBS)
        return b, s0 + sub * R

    def in_copy(i):
        b, r0 = loc(i)
        return pltpu.make_async_copy(
            x_hbm.at[b, pl.ds(r0, R)], bufs[i % NBUF], in_sems[i % NBUF])

    def out_copy(i):
        b, r0 = loc(i)
        return pltpu.make_async_copy(
            bufs[i % NBUF], out_hbm.at[b, pl.ds(r0, R)], out_sems[i % NBUF])

    def pe_copy(sub):
        return pltpu.make_async_copy(
            pt_hbm.at[pl.ds(s0 + sub * R, R)],
            pe_buf.at[pl.ds(sub * R, R)], pe_sems[sub])

    for k in range(NBUF - 1):
        in_copy(k).start()
    for sub in range(SUBS):
        pe_copy(sub).start()

    for i in range(NCHUNKS):
        if i + NBUF - 1 < NCHUNKS:
            if i >= 1:
                out_copy(i - 1).wait()
            in_copy(i + NBUF - 1).start()
        if i < SUBS:
            pe_copy(i).wait()
        in_copy(i).wait()

        buf = bufs[i % NBUF]
        row_base = (i % SUBS) * R

        @plsc.parallel_loop(0, VREGS_PER_CHUNK, unroll=8)
        def _(v):
            r = v >> 6
            coff = (v & (VREGS_PER_ROW - 1)) * LANES
            plsc.addupdate(
                buf.at[r, pl.ds(coff, LANES)],
                pe_buf[row_base + r, pl.ds(coff, LANES)],
            )

        out_copy(i).start()
    for k in range(NBUF):
        if NCHUNKS - NBUF + k >= 0:
            out_copy(NCHUNKS - NBUF + k).wait()


@jax.jit
def kernel(x, pos_table):
    mesh = plsc.VectorSubcoreMesh(core_axis_name="c", subcore_axis_name="s")
    return pl.kernel(
        _sc_body,
        out_type=jax.ShapeDtypeStruct((B, S, D), jnp.float32),
        mesh=mesh,
        scratch_types=(
            [pltpu.VMEM((S_PER_W, D), jnp.float32)]
            + [pltpu.VMEM((R, D), jnp.float32)] * NBUF
            + [pltpu.SemaphoreType.DMA] * (2 * NBUF + SUBS)
        ),
    )(x, pos_table)


# speedup vs baseline: 1.0389x; 1.0389x over previous
import jax
import jax.numpy as jnp
from jax import lax
from jax.experimental import pallas as pl
from jax.experimental.pallas import tpu as pltpu
from jax.experimental.pallas import tpu_sc as plsc

B, S, D = 4, 2048, 1024
NC, NS = 2, 16
NW = NC * NS
S_PER_W = S // NW
R = 16
SUBS = S_PER_W // R
NCHUNKS = B * SUBS
NBUF = 3
LANES = 16
VREGS_PER_ROW = D // LANES
VREGS_PER_CHUNK = R * VREGS_PER_ROW


def _sc_body(x_hbm, pt_hbm, out_hbm, pe_buf, *rest):
    xbufs = rest[:NBUF]
    in_s = rest[NBUF:2 * NBUF]
    out_s = rest[2 * NBUF:3 * NBUF]
    pe_sems = rest[3 * NBUF:]
    cid = lax.axis_index("c")
    sid = lax.axis_index("s")
    wid = sid * NC + cid
    s0 = wid * S_PER_W
    bufs = tuple(xbufs)
    in_sems = tuple(in_s)
    out_sems = tuple(out_s)

    def loc(i):
        b, sub = divmod(i, SUBS)
        return b, s0 + sub * R

    def in_copy(i):
        b, r0 = loc(i)
        return pltpu.make_async_copy(
            x_hbm.at[b, pl.ds(r0, R)], bufs[i % NBUF], in_sems[i % NBUF])

    def out_copy(i):
        b, r0 = loc(i)
        return pltpu.make_async_copy(
            bufs[i % NBUF], out_hbm.at[b, pl.ds(r0, R)], out_sems[i % NBUF])

    def pe_copy(sub):
        return pltpu.make_async_copy(
            pt_hbm.at[pl.ds(s0 + sub * R, R)],
            pe_buf.at[pl.ds(sub * R, R)], pe_sems[sub])

    for k in range(NBUF - 1):
        in_copy(k).start()
    for sub in range(SUBS):
        pe_copy(sub).start()

    for i in range(NCHUNKS):
        if i + NBUF - 1 < NCHUNKS:
            if i >= 1:
                out_copy(i - 1).wait()
            in_copy(i + NBUF - 1).start()
        if i < SUBS:
            pe_copy(i).wait()
        in_copy(i).wait()

        buf = bufs[i % NBUF]
        row_base = (i % SUBS) * R

        @plsc.parallel_loop(0, VREGS_PER_CHUNK, unroll=8)
        def _(v):
            r = v >> 6
            coff = (v & (VREGS_PER_ROW - 1)) * LANES
            plsc.addupdate(
                buf.at[r, pl.ds(coff, LANES)],
                pe_buf[row_base + r, pl.ds(coff, LANES)],
            )

        out_copy(i).start()
    for k in range(NBUF):
        if NCHUNKS - NBUF + k >= 0:
            out_copy(NCHUNKS - NBUF + k).wait()


@jax.jit
def kernel(x, pos_table):
    mesh = plsc.VectorSubcoreMesh(core_axis_name="c", subcore_axis_name="s")
    return pl.kernel(
        _sc_body,
        out_type=jax.ShapeDtypeStruct((B, S, D), jnp.float32),
        mesh=mesh,
        scratch_types=(
            [pltpu.VMEM((S_PER_W, D), jnp.float32)]
            + [pltpu.VMEM((R, D), jnp.float32)] * NBUF
            + [pltpu.SemaphoreType.DMA] * (2 * NBUF + SUBS)
        ),
    )(x, pos_table)
